# Initial kernel scaffold; baseline (speedup 1.0000x reference)
import jax
import jax.numpy as jnp
from jax.experimental import pallas as pl


def kernel(x, targets, table):
    raise NotImplementedError("write your pallas kernel here")


# speedup vs baseline: 1.6183x; 1.6183x over previous
import functools

import jax
import jax.numpy as jnp
from jax import lax
from jax.experimental import pallas as pl
from jax.experimental.pallas import tpu as pltpu
from jax.experimental.pallas import tpu_sc as plsc

V = 1000
NTOK = 51200
_INFO = plsc.get_sparse_core_info()
NC, NS, L = _INFO.num_cores, _INFO.num_subcores, _INFO.num_lanes
NW = NC * NS
PER_W = NTOK // NW
CHUNK = 64
NCHUNK = PER_W // CHUNK


def _lse_body(table_ref, out_ref):
    t = table_ref[...]
    m = jnp.max(t, axis=1, keepdims=True)
    s = jnp.sum(jnp.exp(t - m), axis=1, keepdims=True)
    out_ref[...] = m + jnp.log(s)


def _lse_call(table):
    return pl.pallas_call(
        _lse_body,
        out_shape=jax.ShapeDtypeStruct((V, 1), jnp.float32),
    )(table)


def _sc_body(table_hbm, x_hbm, t_hbm, lse_hbm, out_hbm, part_hbm,
             idx_v, tgt_v, rows_v, lse_v, acc_v, sem):
    wid = lax.axis_index("s") * NC + lax.axis_index("c")
    pltpu.sync_copy(lse_hbm, lse_v)
    acc_v[...] = jnp.zeros((L,), jnp.float32)

    def chunk(c, carry):
        base = wid * PER_W + c * CHUNK
        pltpu.sync_copy(x_hbm.at[pl.ds(base, CHUNK)], idx_v)
        pltpu.sync_copy(t_hbm.at[pl.ds(base, CHUNK)], tgt_v)
        pltpu.async_copy(table_hbm.at[idx_v], rows_v, sem).wait()
        pltpu.sync_copy(rows_v, out_hbm.at[pl.ds(base, CHUNK)])
        for j in range(CHUNK // L):
            x16 = idx_v[pl.ds(j * L, L)]
            t16 = tgt_v[pl.ds(j * L, L)]
            l16 = plsc.load_gather(lse_v, [x16])
            r16 = lax.iota(jnp.int32, L) + (j * L)
            p16 = plsc.load_gather(rows_v, [r16, t16])
            acc_v[...] = acc_v[...] + (l16 - p16)
        return carry

    lax.fori_loop(0, NCHUNK, chunk, 0)
    pltpu.sync_copy(acc_v, part_hbm.at[wid])


_sc_call = functools.partial(
    pl.kernel,
    mesh=plsc.VectorSubcoreMesh(core_axis_name="c", subcore_axis_name="s"),
    compiler_params=pltpu.CompilerParams(
        needs_layout_passes=False, use_tc_tiling_on_sc=False
    ),
    out_type=(
        jax.ShapeDtypeStruct((NTOK, V), jnp.float32),
        jax.ShapeDtypeStruct((NW, L), jnp.float32),
    ),
    scratch_types=[
        pltpu.VMEM((CHUNK,), jnp.int32),
        pltpu.VMEM((CHUNK,), jnp.int32),
        pltpu.VMEM((CHUNK, V), jnp.float32),
        pltpu.VMEM((V,), jnp.float32),
        pltpu.VMEM((L,), jnp.float32),
        pltpu.SemaphoreType.DMA,
    ],
)(_sc_body)


def kernel(x, targets, table):
    xf = x.reshape(-1).astype(jnp.int32)
    tf = targets.reshape(-1).astype(jnp.int32)
    lse = _lse_call(table).reshape(V)
    logits2, parts = _sc_call(table, xf, tf, lse)
    loss = jnp.sum(parts) / jnp.float32(NTOK)
    return (logits2, loss)


# speedup vs baseline: 1.6685x; 1.0310x over previous
import functools

import jax
import jax.numpy as jnp
from jax import lax
from jax.experimental import pallas as pl
from jax.experimental.pallas import tpu as pltpu
from jax.experimental.pallas import tpu_sc as plsc

V = 1000
VP = 1024
NTOK = 51200
_INFO = plsc.get_sparse_core_info()
NC, NS, L = _INFO.num_cores, _INFO.num_subcores, _INFO.num_lanes
NW = NC * NS
PER_W = NTOK // NW
CHUNK = 64
NCHUNK = PER_W // CHUNK


def _lse_body(table_ref, out_ref):
    t = table_ref[...]
    m = jnp.max(t, axis=1, keepdims=True)
    s = jnp.sum(jnp.exp(t - m), axis=1, keepdims=True)
    out_ref[...] = m + jnp.log(s)


def _lse_call(table):
    return pl.pallas_call(
        _lse_body,
        out_shape=jax.ShapeDtypeStruct((V, 1), jnp.float32),
    )(table)


def _sc_body(table_hbm, x_hbm, t_hbm, lse_hbm, out_hbm, part_hbm,
             idx_v, tgt_v, rows_v, lse_v, acc_v, sem):
    wid = lax.axis_index("s") * NC + lax.axis_index("c")
    pltpu.sync_copy(lse_hbm, lse_v)
    acc_v[...] = jnp.zeros((L,), jnp.float32)

    def chunk(c, carry):
        base = wid * PER_W + c * CHUNK
        pltpu.sync_copy(x_hbm.at[pl.ds(base, CHUNK)], idx_v)
        pltpu.sync_copy(t_hbm.at[pl.ds(base, CHUNK)], tgt_v)
        pltpu.async_copy(table_hbm.at[idx_v], rows_v, sem).wait()
        pltpu.sync_copy(rows_v, out_hbm.at[pl.ds(base, CHUNK)])
        for j in range(CHUNK // L):
            x16 = idx_v[pl.ds(j * L, L)]
            t16 = tgt_v[pl.ds(j * L, L)]
            l16 = plsc.load_gather(lse_v, [x16])
            r16 = lax.iota(jnp.int32, L) + (j * L)
            p16 = plsc.load_gather(rows_v, [r16, t16])
            acc_v[...] = acc_v[...] + (l16 - p16)
        return carry

    lax.fori_loop(0, NCHUNK, chunk, 0)
    pltpu.sync_copy(acc_v, part_hbm.at[wid])


_sc_call = functools.partial(
    pl.kernel,
    mesh=plsc.VectorSubcoreMesh(core_axis_name="c", subcore_axis_name="s"),
    compiler_params=pltpu.CompilerParams(
        needs_layout_passes=False, use_tc_tiling_on_sc=False
    ),
    out_type=(
        jax.ShapeDtypeStruct((NTOK, VP), jnp.float32),
        jax.ShapeDtypeStruct((NW, L), jnp.float32),
    ),
    scratch_types=[
        pltpu.VMEM((CHUNK,), jnp.int32),
        pltpu.VMEM((CHUNK,), jnp.int32),
        pltpu.VMEM((CHUNK, VP), jnp.float32),
        pltpu.VMEM((V,), jnp.float32),
        pltpu.VMEM((L,), jnp.float32),
        pltpu.SemaphoreType.DMA,
    ],
)(_sc_body)


def kernel(x, targets, table):
    xf = x.reshape(-1).astype(jnp.int32)
    tf = targets.reshape(-1).astype(jnp.int32)
    table_p = jnp.pad(table, ((0, 0), (0, VP - V)))
    lse = _lse_call(table).reshape(V)
    out_pad, parts = _sc_call(table_p, xf, tf, lse)
    logits2 = out_pad[:, :V]
    loss = jnp.sum(parts) / jnp.float32(NTOK)
    return (logits2, loss)


# speedup vs baseline: 1.6819x; 1.0080x over previous
import functools

import jax
import jax.numpy as jnp
from jax import lax
from jax.experimental import pallas as pl
from jax.experimental.pallas import tpu as pltpu
from jax.experimental.pallas import tpu_sc as plsc

V = 1000
NTOK = 51200
_INFO = plsc.get_sparse_core_info()
NC, NS, L = _INFO.num_cores, _INFO.num_subcores, _INFO.num_lanes
NW = NC * NS
PER_W = NTOK // NW
CHUNK = 32
NCHUNK = PER_W // CHUNK


def _lse_body(table_ref, out_ref):
    t = table_ref[...]
    m = jnp.max(t, axis=1, keepdims=True)
    s = jnp.sum(jnp.exp(t - m), axis=1, keepdims=True)
    out_ref[...] = m + jnp.log(s)


def _lse_call(table):
    return pl.pallas_call(
        _lse_body,
        out_shape=jax.ShapeDtypeStruct((V, 1), jnp.float32),
    )(table)


def _sc_body(table_hbm, x_hbm, t_hbm, lse_hbm, out_hbm, part_hbm,
             idx0, idx1, tgt0, tgt1, rows0, rows1, lse_v, acc_v, g0, g1):
    wid = lax.axis_index("s") * NC + lax.axis_index("c")
    pltpu.sync_copy(lse_hbm, lse_v)
    acc_v[...] = jnp.zeros((L,), jnp.float32)
    base0 = wid * PER_W

    idx = (idx0, idx1)
    tgt = (tgt0, tgt1)
    rows = (rows0, rows1)
    gsem = (g0, g1)

    for b in range(2):
        pltpu.sync_copy(x_hbm.at[pl.ds(base0 + b * CHUNK, CHUNK)], idx[b])
        pltpu.sync_copy(t_hbm.at[pl.ds(base0 + b * CHUNK, CHUNK)], tgt[b])
        pltpu.async_copy(table_hbm.at[idx[b]], rows[b], gsem[b])

    def pair(g, carry):
        for b in range(2):
            c = g * 2 + b
            basec = base0 + c * CHUNK
            pltpu.make_async_copy(table_hbm.at[idx[b]], rows[b], gsem[b]).wait()
            pltpu.sync_copy(rows[b], out_hbm.at[pl.ds(basec, CHUNK)])
            for j in range(CHUNK // L):
                x16 = idx[b][pl.ds(j * L, L)]
                t16 = tgt[b][pl.ds(j * L, L)]
                l16 = plsc.load_gather(lse_v, [x16])
                r16 = lax.iota(jnp.int32, L) + (j * L)
                p16 = plsc.load_gather(rows[b], [r16, t16])
                acc_v[...] = acc_v[...] + (l16 - p16)

            @pl.when(c + 2 < NCHUNK)
            def _():
                nxt = basec + 2 * CHUNK
                pltpu.sync_copy(x_hbm.at[pl.ds(nxt, CHUNK)], idx[b])
                pltpu.sync_copy(t_hbm.at[pl.ds(nxt, CHUNK)], tgt[b])
                pltpu.async_copy(table_hbm.at[idx[b]], rows[b], gsem[b])

        return carry

    lax.fori_loop(0, NCHUNK // 2, pair, 0)
    pltpu.sync_copy(acc_v, part_hbm.at[wid])


_sc_call = functools.partial(
    pl.kernel,
    mesh=plsc.VectorSubcoreMesh(core_axis_name="c", subcore_axis_name="s"),
    compiler_params=pltpu.CompilerParams(
        needs_layout_passes=False, use_tc_tiling_on_sc=False
    ),
    out_type=(
        jax.ShapeDtypeStruct((NTOK, V), jnp.float32),
        jax.ShapeDtypeStruct((NW, L), jnp.float32),
    ),
    scratch_types=[
        pltpu.VMEM((CHUNK,), jnp.int32),
        pltpu.VMEM((CHUNK,), jnp.int32),
        pltpu.VMEM((CHUNK,), jnp.int32),
        pltpu.VMEM((CHUNK,), jnp.int32),
        pltpu.VMEM((CHUNK, V), jnp.float32),
        pltpu.VMEM((CHUNK, V), jnp.float32),
        pltpu.VMEM((V,), jnp.float32),
        pltpu.VMEM((L,), jnp.float32),
        pltpu.SemaphoreType.DMA,
        pltpu.SemaphoreType.DMA,
    ],
)(_sc_body)


def kernel(x, targets, table):
    xf = x.reshape(-1).astype(jnp.int32)
    tf = targets.reshape(-1).astype(jnp.int32)
    lse = _lse_call(table).reshape(V)
    logits2, parts = _sc_call(table, xf, tf, lse)
    loss = jnp.sum(parts) / jnp.float32(NTOK)
    return (logits2, loss)


# speedup vs baseline: 1.7575x; 1.0450x over previous
import functools

import jax
import jax.numpy as jnp
from jax import lax
from jax.experimental import pallas as pl
from jax.experimental.pallas import tpu as pltpu
from jax.experimental.pallas import tpu_sc as plsc

V = 1000
VP = 1024
NTOK = 51200
_INFO = plsc.get_sparse_core_info()
NC, NS, L = _INFO.num_cores, _INFO.num_subcores, _INFO.num_lanes
NW = NC * NS
PER_W = NTOK // NW
CHUNK = 32
NCHUNK = PER_W // CHUNK
BT = 512


def _lse_body(table_ref, out_ref):
    t = table_ref[...]
    m = jnp.max(t, axis=1, keepdims=True)
    s = jnp.sum(jnp.exp(t - m), axis=1, keepdims=True)
    out_ref[...] = m + jnp.log(s)


def _lse_call(table):
    return pl.pallas_call(
        _lse_body,
        out_shape=jax.ShapeDtypeStruct((V, 1), jnp.float32),
    )(table)


def _detile_body(in_ref, out_ref):
    v = in_ref[...]
    out_ref[...] = v.reshape(BT, VP)[:, :V]


def _detile_call(out3):
    return pl.pallas_call(
        _detile_body,
        grid=(NTOK // BT,),
        in_specs=[pl.BlockSpec((BT, 8, 128), lambda i: (i, 0, 0))],
        out_specs=pl.BlockSpec((BT, V), lambda i: (i, 0)),
        out_shape=jax.ShapeDtypeStruct((NTOK, V), jnp.float32),
    )(out3)


def _sc_body(table_hbm, flat_hbm, x_hbm, t_hbm, lse_hbm, out_hbm, part_hbm,
             idx0, idx1, tgt0, tgt1, rows0, rows1, fid0, fid1, pick0, pick1,
             lse_v, acc_v, g0, g1, p0, p1):
    wid = lax.axis_index("s") * NC + lax.axis_index("c")
    pltpu.sync_copy(lse_hbm, lse_v)
    acc_v[...] = jnp.zeros((L,), jnp.float32)
    base0 = wid * PER_W

    idx = (idx0, idx1)
    tgt = (tgt0, tgt1)
    rows = (rows0, rows1)
    fid = (fid0, fid1)
    pick = (pick0, pick1)
    gsem = (g0, g1)
    psem = (p0, p1)

    def start(b, basec):
        pltpu.sync_copy(x_hbm.at[pl.ds(basec, CHUNK)], idx[b])
        pltpu.sync_copy(t_hbm.at[pl.ds(basec, CHUNK)], tgt[b])
        pltpu.async_copy(table_hbm.at[idx[b]], rows[b], gsem[b])
        for j in range(CHUNK // L):
            s = pl.ds(j * L, L)
            fid[b][s] = idx[b][s] * V + tgt[b][s]
        pltpu.async_copy(flat_hbm.at[fid[b]], pick[b], psem[b])

    for b in range(2):
        start(b, base0 + b * CHUNK)

    def pair(g, carry):
        for b in range(2):
            c = g * 2 + b
            basec = base0 + c * CHUNK
            pltpu.make_async_copy(table_hbm.at[idx[b]], rows[b], gsem[b]).wait()
            pltpu.sync_copy(rows[b], out_hbm.at[pl.ds(basec, CHUNK)])
            pltpu.make_async_copy(flat_hbm.at[fid[b]], pick[b], psem[b]).wait()
            for j in range(CHUNK // L):
                s = pl.ds(j * L, L)
                l16 = plsc.load_gather(lse_v, [idx[b][s]])
                acc_v[...] = acc_v[...] + (l16 - pick[b][s])

            @pl.when(c + 2 < NCHUNK)
            def _():
                start(b, basec + 2 * CHUNK)

        return carry

    lax.fori_loop(0, NCHUNK // 2, pair, 0)
    pltpu.sync_copy(acc_v, part_hbm.at[pl.ds(wid * L, L)])


_sc_call = functools.partial(
    pl.kernel,
    mesh=plsc.VectorSubcoreMesh(core_axis_name="c", subcore_axis_name="s"),
    compiler_params=pltpu.CompilerParams(
        needs_layout_passes=False, use_tc_tiling_on_sc=False
    ),
    out_type=(
        jax.ShapeDtypeStruct((NTOK, 8, 128), jnp.float32),
        jax.ShapeDtypeStruct((NW * L,), jnp.float32),
    ),
    scratch_types=[
        pltpu.VMEM((CHUNK,), jnp.int32),
        pltpu.VMEM((CHUNK,), jnp.int32),
        pltpu.VMEM((CHUNK,), jnp.int32),
        pltpu.VMEM((CHUNK,), jnp.int32),
        pltpu.VMEM((CHUNK, 8, 128), jnp.float32),
        pltpu.VMEM((CHUNK, 8, 128), jnp.float32),
        pltpu.VMEM((CHUNK,), jnp.int32),
        pltpu.VMEM((CHUNK,), jnp.int32),
        pltpu.VMEM((CHUNK,), jnp.float32),
        pltpu.VMEM((CHUNK,), jnp.float32),
        pltpu.VMEM((V,), jnp.float32),
        pltpu.VMEM((L,), jnp.float32),
        pltpu.SemaphoreType.DMA,
        pltpu.SemaphoreType.DMA,
        pltpu.SemaphoreType.DMA,
        pltpu.SemaphoreType.DMA,
    ],
)(_sc_body)


def kernel(x, targets, table):
    xf = x.reshape(-1).astype(jnp.int32)
    tf = targets.reshape(-1).astype(jnp.int32)
    table_p = jnp.pad(table, ((0, 0), (0, VP - V)))
    table3 = table_p.reshape(V, 8, 128)
    flat = table.reshape(-1)
    lse = _lse_call(table).reshape(V)
    out3, parts = _sc_call(table3, flat, xf, tf, lse)
    logits2 = _detile_call(out3)
    loss = jnp.sum(parts) / jnp.float32(NTOK)
    return (logits2, loss)


# speedup vs baseline: 1.9591x; 1.1147x over previous
import functools

import jax
import jax.numpy as jnp
from jax import lax
from jax.experimental import pallas as pl
from jax.experimental.pallas import tpu as pltpu
from jax.experimental.pallas import tpu_sc as plsc

V = 1000
VP = 1024
NTOK = 51200
_INFO = plsc.get_sparse_core_info()
NC, NS, L = _INFO.num_cores, _INFO.num_subcores, _INFO.num_lanes
NW = NC * NS
PER_W = NTOK // NW
CHUNK = 32
NCHUNK = PER_W // CHUNK
BT = 512


def _lse_body(table_ref, out_ref):
    t = table_ref[...]
    m = jnp.max(t, axis=1, keepdims=True)
    s = jnp.sum(jnp.exp(t - m), axis=1, keepdims=True)
    out_ref[...] = m + jnp.log(s)


def _lse_call(table):
    return pl.pallas_call(
        _lse_body,
        out_shape=jax.ShapeDtypeStruct((V, 1), jnp.float32),
    )(table)


def _detile_body(in_ref, out_ref):
    v = in_ref[...]
    out_ref[...] = v.reshape(BT, VP)[:, :V]


def _detile_call(out3):
    return pl.pallas_call(
        _detile_body,
        grid=(NTOK // BT,),
        in_specs=[pl.BlockSpec((BT, 8, 128), lambda i: (i, 0, 0))],
        out_specs=pl.BlockSpec((BT, V), lambda i: (i, 0)),
        out_shape=jax.ShapeDtypeStruct((NTOK, V), jnp.float32),
    )(out3)


def _sc_body(table_hbm, flat_hbm, x_hbm, t_hbm, lse_hbm, out_hbm, part_hbm,
             idx0, idx1, tgt0, tgt1, rows0, rows1, fid0, fid1, pick0, pick1,
             lse_v, acc_v, g0, g1, p0, p1):
    wid = lax.axis_index("s") * NC + lax.axis_index("c")
    pltpu.sync_copy(lse_hbm, lse_v)
    acc_v[...] = jnp.zeros((L,), jnp.float32)
    base0 = wid * PER_W

    idx = (idx0, idx1)
    tgt = (tgt0, tgt1)
    rows = (rows0, rows1)
    fid = (fid0, fid1)
    pick = (pick0, pick1)
    gsem = (g0, g1)
    psem = (p0, p1)

    def start(b, basec):
        pltpu.sync_copy(x_hbm.at[pl.ds(basec, CHUNK)], idx[b])
        pltpu.sync_copy(t_hbm.at[pl.ds(basec, CHUNK)], tgt[b])
        pltpu.async_copy(table_hbm.at[idx[b]], rows[b], gsem[b])
        for j in range(CHUNK // L):
            s = pl.ds(j * L, L)
            fid[b][s] = idx[b][s] * V + tgt[b][s]
        pltpu.async_copy(flat_hbm.at[fid[b]], pick[b], psem[b])

    for b in range(2):
        start(b, base0 + b * CHUNK)

    def pair(g, carry):
        for b in range(2):
            c = g * 2 + b
            basec = base0 + c * CHUNK
            pltpu.make_async_copy(table_hbm.at[idx[b]], rows[b], gsem[b]).wait()
            pltpu.sync_copy(rows[b], out_hbm.at[pl.ds(basec, CHUNK)])
            pltpu.make_async_copy(flat_hbm.at[fid[b]], pick[b], psem[b]).wait()
            for j in range(CHUNK // L):
                s = pl.ds(j * L, L)
                l16 = plsc.load_gather(lse_v, [idx[b][s]])
                acc_v[...] = acc_v[...] + (l16 - pick[b][s])

            @pl.when(c + 2 < NCHUNK)
            def _():
                start(b, basec + 2 * CHUNK)

        return carry

    lax.fori_loop(0, NCHUNK // 2, pair, 0)
    pltpu.sync_copy(acc_v, part_hbm.at[pl.ds(wid * L, L)])


_sc_call = functools.partial(
    pl.kernel,
    mesh=plsc.VectorSubcoreMesh(core_axis_name="c", subcore_axis_name="s"),
    compiler_params=pltpu.CompilerParams(
        needs_layout_passes=False, use_tc_tiling_on_sc=False
    ),
    out_type=(
        jax.ShapeDtypeStruct((NTOK, 8, 128), jnp.float32),
        jax.ShapeDtypeStruct((NW * L,), jnp.float32),
    ),
    scratch_types=[
        pltpu.VMEM((CHUNK,), jnp.int32),
        pltpu.VMEM((CHUNK,), jnp.int32),
        pltpu.VMEM((CHUNK,), jnp.int32),
        pltpu.VMEM((CHUNK,), jnp.int32),
        pltpu.VMEM((CHUNK, 8, 128), jnp.float32),
        pltpu.VMEM((CHUNK, 8, 128), jnp.float32),
        pltpu.VMEM((CHUNK,), jnp.int32),
        pltpu.VMEM((CHUNK,), jnp.int32),
        pltpu.VMEM((CHUNK,), jnp.float32),
        pltpu.VMEM((CHUNK,), jnp.float32),
        pltpu.VMEM((V,), jnp.float32),
        pltpu.VMEM((L,), jnp.float32),
        pltpu.SemaphoreType.DMA,
        pltpu.SemaphoreType.DMA,
        pltpu.SemaphoreType.DMA,
        pltpu.SemaphoreType.DMA,
    ],
)(_sc_body)


def kernel(x, targets, table):
    xf = x.reshape(-1).astype(jnp.int32)
    tf = targets.reshape(-1).astype(jnp.int32)
    table_p = jnp.pad(table, ((0, 0), (0, VP - V)))
    table3 = table_p.reshape(V, 8, 128)
    flat = table.reshape(-1)
    lse = _lse_call(table).reshape(V)
    out3, parts = _sc_call(table3, flat, xf, tf, lse)
    logits2 = out3.reshape(NTOK, VP)[:, :V]
    loss = jnp.sum(parts) / jnp.float32(NTOK)
    return (logits2, loss)
